# baseline (speedup 1.0000x reference)
import functools

import jax
import jax.numpy as jnp
from jax.experimental import pallas as pl

N_NODES = 50000
N_EDGES = 800000
IN_CH = 54
HID = 32
OUT_CH = 64
EDGE_DIM = 4
NUM_MOLS = 2048
NUM_TIMESTEPS = 2


def _leaky(x):
    return jnp.where(x > 0, x, 0.01 * x)


def _seg_softmax(alpha, index, num_segments):
    amax = jax.ops.segment_max(alpha, index, num_segments=num_segments)
    amax = jnp.where(jnp.isfinite(amax), amax, 0.0)
    e = jnp.exp(alpha - amax[index])
    den = jax.ops.segment_sum(e, index, num_segments=num_segments)
    return e / (den[index] + 1e-16)


def _gru(x, h, wih, whh, bih, bhh):
    gi = x @ wih.T + bih
    gh = h @ whh.T + bhh
    ir, iz, inn = jnp.split(gi, 3, axis=-1)
    hr, hz, hn = jnp.split(gh, 3, axis=-1)
    r = jax.nn.sigmoid(ir + hr)
    z = jax.nn.sigmoid(iz + hz)
    n = jnp.tanh(inn + r * hn)
    return (1.0 - z) * n + z * h


def _final_body(out_ref, w_ref, b_ref, y_ref):
    out = out_ref[...]
    y = out @ w_ref[...].T + b_ref[...][None, :]
    sp = jnp.log1p(jnp.exp(y))
    y_ref[...] = y * jnp.tanh(sp)


def _final_stage(out, lin2_w, lin2_b):
    return pl.pallas_call(
        _final_body,
        out_shape=jax.ShapeDtypeStruct((NUM_MOLS, OUT_CH), jnp.float32),
    )(out, lin2_w, lin2_b)


def kernel(x, edge_index, edge_attr, batch_index, lin1_w, lin1_b, gate_lin1_w,
           gate_lin2_w, gate_att_l, gate_att_r, gate_bias, gru0_wih, gru0_whh,
           gru0_bih, gru0_bhh, conv1_w, conv1_att_src, conv1_att_dst,
           conv1_bias, gru1_wih, gru1_whh, gru1_bih, gru1_bhh, mol_w,
           mol_att_src, mol_att_dst, mol_bias, molgru_wih, molgru_whh,
           molgru_bih, molgru_bhh, lin2_w, lin2_b):
    src, dst = edge_index[0], edge_index[1]
    x = _leaky(x @ lin1_w.T + lin1_b)
    m = _leaky(jnp.concatenate([x[src], edge_attr], axis=-1) @ gate_lin1_w.T)
    alpha = _leaky((m * gate_att_l).sum(-1) + (x[dst] * gate_att_r).sum(-1))
    alpha = _seg_softmax(alpha, dst, N_NODES)
    h = jax.ops.segment_sum((m @ gate_lin2_w.T) * alpha[:, None], dst,
                            num_segments=N_NODES) + gate_bias
    h = jax.nn.elu(h)
    x = jax.nn.relu(_gru(h, x, gru0_wih, gru0_whh, gru0_bih, gru0_bhh))
    xp = x @ conv1_w.T
    a = _leaky((xp * conv1_att_src).sum(-1)[src] + (xp * conv1_att_dst).sum(-1)[dst])
    a = _seg_softmax(a, dst, N_NODES)
    h = jax.ops.segment_sum(xp[src] * a[:, None], dst, num_segments=N_NODES) + conv1_bias
    h = jax.nn.relu(h)
    x = jax.nn.relu(_gru(h, x, gru1_wih, gru1_whh, gru1_bih, gru1_bhh))
    out = jax.nn.relu(jax.ops.segment_sum(x, batch_index, num_segments=NUM_MOLS))
    for _ in range(NUM_TIMESTEPS):
        xs = x @ mol_w.T
        od = out @ mol_w.T
        a = _leaky((xs * mol_att_src).sum(-1) + (od * mol_att_dst).sum(-1)[batch_index])
        a = _seg_softmax(a, batch_index, NUM_MOLS)
        h = jax.ops.segment_sum(xs * a[:, None], batch_index, num_segments=NUM_MOLS) + mol_bias
        h = jax.nn.elu(h)
        out = jax.nn.relu(_gru(h, out, molgru_wih, molgru_whh, molgru_bih, molgru_bhh))
    return _final_stage(out, lin2_w, lin2_b)


# speedup vs baseline: 12.7023x; 12.7023x over previous
import functools

import jax
import jax.numpy as jnp
from jax import lax
from jax.experimental import pallas as pl
from jax.experimental.pallas import tpu as pltpu
from jax.experimental.pallas import tpu_sc as plsc

F32 = jnp.float32
I32 = jnp.int32

N_NODES = 50000
N_EDGES = 800000
IN_CH = 54
HID = 32
OUT_CH = 64
EDGE_DIM = 4
NUM_MOLS = 2048

NC = 2
NS = 16
NW = NC * NS
C = 128

E2 = 802816
EROWS = E2 // C
RPW_E = EROWS // NW
J_E = 14
STEPS_E = RPW_E // J_E

NP = 53248
NROWS_B = NP // C
J_N = 13

NODE_BLK = 2048
NB_N = NP // NODE_BLK
EDGE_BLK = 2048
NB_E = E2 // EDGE_BLK

_mesh = plsc.VectorSubcoreMesh(core_axis_name="c", subcore_axis_name="s")


def _wid():
    return lax.axis_index("s") * NC + lax.axis_index("c")


def _zero_vmem_2d(ref, rows):
    z = jnp.zeros((16,), F32)

    def body(i, _):
        ref[i, pl.ds(0, 16)] = z
        ref[i, pl.ds(16, 16)] = z
        return 0

    lax.fori_loop(0, rows, body, 0)


def _zero_vmem_1d(ref, n):
    z = jnp.zeros((16,), F32)

    def body(i, _):
        ref[pl.ds(i * 16, 16)] = z
        return 0

    lax.fori_loop(0, n // 16, body, 0)


def _stage_idx2(idx1d, idx2d, r, j_chunks, sem):
    cps = []
    for j in range(j_chunks):
        cps.append(pltpu.async_copy(
            idx1d.at[pl.ds((r + j) * C, C)], idx2d.at[j], sem))
    for cp in cps:
        cp.wait()


@functools.partial(
    pl.kernel,
    out_type=(jax.ShapeDtypeStruct((E2, HID), F32),
              jax.ShapeDtypeStruct((E2,), F32)),
    mesh=_mesh,
    compiler_params=pltpu.CompilerParams(use_tc_tiling_on_sc=False),
    scratch_types=[
        pltpu.VMEM((J_E, C), I32),
        pltpu.VMEM((J_E, C), I32),
        pltpu.VMEM((J_E * C, HID), F32),
        pltpu.VMEM((J_E * C,), F32),
        pltpu.SemaphoreType.DMA,
        pltpu.SemaphoreType.DMA,
    ],
)
def _gate_gather(g_tab, r_tab, srcb, dstb, gsrc_out, rdst_out,
                 sidx, didx, rows, rvals, sem1, sem2):
    row0 = _wid() * RPW_E

    def step(t, _):
        r = row0 + t * J_E
        _stage_idx2(srcb, sidx, r, J_E, sem1)
        _stage_idx2(dstb, didx, r, J_E, sem2)
        cps = []
        for j in range(J_E):
            cps.append(pltpu.async_copy(
                g_tab.at[sidx.at[j]], rows.at[pl.ds(j * C, C)], sem1))
            cps.append(pltpu.async_copy(
                r_tab.at[didx.at[j]], rvals.at[pl.ds(j * C, C)], sem2))
        for cp in cps:
            cp.wait()
        pltpu.sync_copy(rows, gsrc_out.at[pl.ds(r * C, J_E * C)])
        pltpu.sync_copy(rvals, rdst_out.at[pl.ds(r * C, J_E * C)])
        return 0

    lax.fori_loop(0, STEPS_E, step, 0)


@functools.partial(
    pl.kernel,
    out_type=(jax.ShapeDtypeStruct((E2, HID), F32),
              jax.ShapeDtypeStruct((E2,), F32),
              jax.ShapeDtypeStruct((E2,), F32)),
    mesh=_mesh,
    compiler_params=pltpu.CompilerParams(use_tc_tiling_on_sc=False),
    scratch_types=[
        pltpu.VMEM((J_E, C), I32),
        pltpu.VMEM((J_E, C), I32),
        pltpu.VMEM((J_E * C, HID), F32),
        pltpu.VMEM((J_E * C,), F32),
        pltpu.VMEM((J_E * C,), F32),
        pltpu.SemaphoreType.DMA,
        pltpu.SemaphoreType.DMA,
    ],
)
def _conv_gather(xp_tab, s_tab, d_tab, srcb, dstb,
                 xps_out, ss_out, dd_out,
                 sidx, didx, rows, svals, dvals, sem1, sem2):
    row0 = _wid() * RPW_E

    def step(t, _):
        r = row0 + t * J_E
        _stage_idx2(srcb, sidx, r, J_E, sem1)
        _stage_idx2(dstb, didx, r, J_E, sem2)
        cps = []
        for j in range(J_E):
            cps.append(pltpu.async_copy(
                xp_tab.at[sidx.at[j]], rows.at[pl.ds(j * C, C)], sem1))
            cps.append(pltpu.async_copy(
                s_tab.at[sidx.at[j]], svals.at[pl.ds(j * C, C)], sem2))
            cps.append(pltpu.async_copy(
                d_tab.at[didx.at[j]], dvals.at[pl.ds(j * C, C)], sem2))
        for cp in cps:
            cp.wait()
        pltpu.sync_copy(rows, xps_out.at[pl.ds(r * C, J_E * C)])
        pltpu.sync_copy(svals, ss_out.at[pl.ds(r * C, J_E * C)])
        pltpu.sync_copy(dvals, dd_out.at[pl.ds(r * C, J_E * C)])
        return 0

    lax.fori_loop(0, STEPS_E, step, 0)


@functools.partial(
    pl.kernel,
    out_type=jax.ShapeDtypeStruct((NP,), F32),
    mesh=_mesh,
    compiler_params=pltpu.CompilerParams(use_tc_tiling_on_sc=False),
    scratch_types=[
        pltpu.VMEM((J_N, C), I32),
        pltpu.VMEM((J_N * C,), F32),
        pltpu.SemaphoreType.DMA,
    ],
)
def _mol_gather(od_tab, batchb, odb_out, bidx, vals, sem):
    r = _wid() * J_N
    _stage_idx2(batchb, bidx, r, J_N, sem)
    cps = []
    for j in range(J_N):
        cps.append(pltpu.async_copy(
            od_tab.at[bidx.at[j]], vals.at[pl.ds(j * C, C)], sem))
    for cp in cps:
        cp.wait()
    pltpu.sync_copy(vals, odb_out.at[pl.ds(r * C, J_N * C)])


def _build_scatter(nt, rpw, j_chunks, with_scal):
    steps = rpw // j_chunks
    zt = nt // NS
    zb = min(zt, 128)
    n_zero = zt // zb

    outs = [jax.ShapeDtypeStruct((2 * nt, HID), F32)]
    scratch = [
        pltpu.VMEM((j_chunks, C), I32),
        pltpu.VMEM((j_chunks * C, HID), F32),
        pltpu.VMEM((zb, HID), F32),
        pltpu.VMEM_SHARED((nt, HID), F32),
        pltpu.SemaphoreType.DMA,
    ]
    if with_scal:
        outs.append(jax.ShapeDtypeStruct((2 * nt,), F32))
        scratch += [
            pltpu.VMEM((j_chunks * C,), F32),
            pltpu.VMEM((zt,), F32),
            pltpu.VMEM_SHARED((nt,), F32),
            pltpu.SemaphoreType.DMA,
        ]

    def common(idxb, vals_in, scal_in, rows_out, den_out,
               didx, vbuf, zbuf, acc, sem1, sbuf, dzbuf, dacc, sem2):
        sid = lax.axis_index("s")
        cid = lax.axis_index("c")
        row0 = (sid * NC + cid) * rpw

        _zero_vmem_2d(zbuf, zb)

        def zcopy(k, _):
            pltpu.sync_copy(zbuf, acc.at[pl.ds(sid * zt + k * zb, zb)])
            return 0

        lax.fori_loop(0, n_zero, zcopy, 0)
        if with_scal:
            _zero_vmem_1d(dzbuf, zt)
            pltpu.sync_copy(dzbuf, dacc.at[pl.ds(sid * zt, zt)])
        plsc.subcore_barrier()

        def step(t, _):
            r = row0 + t * j_chunks
            _stage_idx2(idxb, didx, r, j_chunks, sem1)
            pltpu.sync_copy(vals_in.at[pl.ds(r * C, j_chunks * C)], vbuf)
            if with_scal:
                pltpu.sync_copy(scal_in.at[pl.ds(r * C, j_chunks * C)], sbuf)
            cps = []
            for j in range(j_chunks):
                cps.append(pltpu.async_copy(
                    vbuf.at[pl.ds(j * C, C)], acc.at[didx.at[j]], sem1,
                    add=True))
                if with_scal:
                    cps.append(pltpu.async_copy(
                        sbuf.at[pl.ds(j * C, C)], dacc.at[didx.at[j]], sem2,
                        add=True))
            for cp in cps:
                cp.wait()
            return 0

        lax.fori_loop(0, steps, step, 0)
        plsc.subcore_barrier()

        pltpu.sync_copy(acc.at[pl.ds(sid * zt, zt)],
                        rows_out.at[pl.ds(cid * nt + sid * zt, zt)])
        if with_scal:
            pltpu.sync_copy(dacc.at[pl.ds(sid * zt, zt)],
                            den_out.at[pl.ds(cid * nt + sid * zt, zt)])

    if with_scal:
        def body(idxb, vals_in, scal_in, rows_out, den_out,
                 didx, vbuf, zbuf, acc, sem1, sbuf, dzbuf, dacc, sem2):
            common(idxb, vals_in, scal_in, rows_out, den_out,
                   didx, vbuf, zbuf, acc, sem1, sbuf, dzbuf, dacc, sem2)
    else:
        def body(idxb, vals_in, rows_out, didx, vbuf, zbuf, acc, sem1):
            common(idxb, vals_in, None, rows_out, None,
                   didx, vbuf, zbuf, acc, sem1, None, None, None, None)

    return pl.kernel(
        body, out_type=tuple(outs) if len(outs) > 1 else outs[0],
        mesh=_mesh, scratch_types=scratch,
        compiler_params=pltpu.CompilerParams(use_tc_tiling_on_sc=False))


NTH = NP // 2
ACC_R = NTH + C
ZPT = ACC_R // NS
DPT = NTH // NS
RPT_E = EROWS // NS
STEPS_E2 = RPT_E // J_E


@functools.partial(
    pl.kernel,
    out_type=(jax.ShapeDtypeStruct((NP, HID), F32),
              jax.ShapeDtypeStruct((NP,), F32)),
    mesh=_mesh,
    compiler_params=pltpu.CompilerParams(use_tc_tiling_on_sc=False),
    scratch_types=[
        pltpu.VMEM((J_E, C), I32),
        pltpu.VMEM((J_E, C), I32),
        pltpu.VMEM((J_E * C, HID), F32),
        pltpu.VMEM((J_E * C,), F32),
        pltpu.VMEM((ZPT // 8, HID), F32),
        pltpu.VMEM((ZPT,), F32),
        pltpu.VMEM_SHARED((ACC_R, HID), F32),
        pltpu.VMEM_SHARED((ACC_R,), F32),
        pltpu.SemaphoreType.DMA,
        pltpu.SemaphoreType.DMA,
    ],
)
def _edge_scatter(dstb, wm_in, w_in, hnum_out, den_out,
                  didx, lidx, vbuf, sbuf, zbuf, dzbuf, acc, dacc,
                  sem1, sem2):
    sid = lax.axis_index("s")
    cid = lax.axis_index("c")
    base = cid * NTH

    _zero_vmem_2d(zbuf, ZPT // 8)
    _zero_vmem_1d(dzbuf, ZPT)

    def zcopy(k, _):
        pltpu.sync_copy(zbuf, acc.at[pl.ds(sid * ZPT + k * (ZPT // 8),
                                           ZPT // 8)])
        return 0

    lax.fori_loop(0, 8, zcopy, 0)
    pltpu.sync_copy(dzbuf, dacc.at[pl.ds(sid * ZPT, ZPT)])
    plsc.subcore_barrier()

    row0 = sid * RPT_E

    def step(t, _):
        r = row0 + t * J_E
        _stage_idx2(dstb, didx, r, J_E, sem1)
        for j in range(J_E):
            for v in range(C // 16):
                d = didx[j, pl.ds(v * 16, 16)]
                loc = d - base
                ok = (loc >= 0) & (loc < NTH)
                dummy = NTH + (d & (C - 1))
                lidx[j, pl.ds(v * 16, 16)] = jnp.where(ok, loc, dummy)
        pltpu.sync_copy(wm_in.at[pl.ds(r * C, J_E * C)], vbuf)
        pltpu.sync_copy(w_in.at[pl.ds(r * C, J_E * C)], sbuf)
        cps = []
        for j in range(J_E):
            cps.append(pltpu.async_copy(
                vbuf.at[pl.ds(j * C, C)], acc.at[lidx.at[j]], sem1,
                add=True))
            cps.append(pltpu.async_copy(
                sbuf.at[pl.ds(j * C, C)], dacc.at[lidx.at[j]], sem2,
                add=True))
        for cp in cps:
            cp.wait()
        return 0

    lax.fori_loop(0, STEPS_E2, step, 0)
    plsc.subcore_barrier()

    pltpu.sync_copy(acc.at[pl.ds(sid * DPT, DPT)],
                    hnum_out.at[pl.ds(cid * NTH + sid * DPT, DPT)])
    pltpu.sync_copy(dacc.at[pl.ds(sid * DPT, DPT)],
                    den_out.at[pl.ds(cid * NTH + sid * DPT, DPT)])


_pool_scatter = _build_scatter(NUM_MOLS, J_N, J_N, with_scal=False)
_mol_scatter = _build_scatter(NUM_MOLS, J_N, J_N, with_scal=True)


def _leaky(x):
    return jnp.where(x > 0, x, 0.01 * x)


def _elu(x):
    return jnp.where(x > 0, x, jnp.exp(x) - 1.0)


def _gru_tc(x, h, wihT, whhT, bih, bhh):
    gi = x @ wihT + bih
    gh = h @ whhT + bhh
    r = jax.nn.sigmoid(gi[:, :HID] + gh[:, :HID])
    z = jax.nn.sigmoid(gi[:, HID:2 * HID] + gh[:, HID:2 * HID])
    n = jnp.tanh(gi[:, 2 * HID:] + r * gh[:, 2 * HID:])
    return (1.0 - z) * n + z * h


def _s1_body(x_ref, w1T_ref, b1_ref, wxT_ref, attr_ref,
             x1_ref, g_ref, r_ref):
    x1 = _leaky(x_ref[...] @ w1T_ref[...] + b1_ref[...])
    x1_ref[...] = x1
    g_ref[...] = x1 @ wxT_ref[...]
    r_ref[...] = jnp.sum(x1 * attr_ref[...], axis=1)


def _s3_body(gsrc_ref, ea0_ref, ea1_ref, ea2_ref, ea3_ref, rdst_ref,
             we_ref, attl_ref, wm_ref, w_ref):
    i = pl.program_id(0)
    eterm = (ea0_ref[...][:, None] * we_ref[...][0:1, :]
             + ea1_ref[...][:, None] * we_ref[...][1:2, :]
             + ea2_ref[...][:, None] * we_ref[...][2:3, :]
             + ea3_ref[...][:, None] * we_ref[...][3:4, :])
    m = _leaky(gsrc_ref[...] + eterm)
    p = jnp.sum(m * attl_ref[...], axis=1)
    alpha = _leaky(p + rdst_ref[...])
    wv = jnp.exp(alpha)
    rowid = i * EDGE_BLK + lax.broadcasted_iota(I32, (EDGE_BLK,), 0)
    wv = jnp.where(rowid < N_EDGES, wv, 0.0)
    w_ref[...] = wv
    wm_ref[...] = m * wv[:, None]


def _s5_body(h0_ref, d0_ref, x1_ref,
             l2T_ref, gb_ref, wihT_ref, whhT_ref, bih_ref, bhh_ref,
             cwT_ref, atts_ref, attd_ref,
             x2_ref, xp_ref, s1_ref, d1o_ref):
    hn = h0_ref[...]
    dn = d0_ref[...]
    hmean = jnp.where(dn[:, None] > 0, hn / (dn[:, None] + 1e-30), 0.0)
    h = _elu(hmean @ l2T_ref[...] + gb_ref[...])
    x2 = jax.nn.relu(_gru_tc(h, x1_ref[...], wihT_ref[...], whhT_ref[...],
                             bih_ref[...], bhh_ref[...]))
    x2_ref[...] = x2
    xp = x2 @ cwT_ref[...]
    xp_ref[...] = xp
    s1_ref[...] = jnp.sum(xp * atts_ref[...], axis=1)
    d1o_ref[...] = jnp.sum(xp * attd_ref[...], axis=1)


def _s7_body(ss_ref, dd_ref, xps_ref, wxp_ref, w_ref):
    i = pl.program_id(0)
    wv = jnp.exp(_leaky(ss_ref[...] + dd_ref[...]))
    rowid = i * EDGE_BLK + lax.broadcasted_iota(I32, (EDGE_BLK,), 0)
    wv = jnp.where(rowid < N_EDGES, wv, 0.0)
    w_ref[...] = wv
    wxp_ref[...] = xps_ref[...] * wv[:, None]


def _s9_body(h0_ref, d0_ref, x2_ref,
             cb_ref, wihT_ref, whhT_ref, bih_ref, bhh_ref,
             mwT_ref, matts_ref,
             x3_ref, xs_ref, smol_ref):
    i = pl.program_id(0)
    hn = h0_ref[...]
    dn = d0_ref[...]
    hmean = jnp.where(dn[:, None] > 0, hn / (dn[:, None] + 1e-30), 0.0)
    h = jax.nn.relu(hmean + cb_ref[...])
    x3 = jax.nn.relu(_gru_tc(h, x2_ref[...], wihT_ref[...], whhT_ref[...],
                             bih_ref[...], bhh_ref[...]))
    rowid = i * NODE_BLK + lax.broadcasted_iota(I32, (NODE_BLK,), 0)
    rowmask = jnp.where(rowid < N_NODES, 1.0, 0.0)
    x3 = x3 * rowmask[:, None]
    x3_ref[...] = x3
    xs = x3 @ mwT_ref[...]
    xs_ref[...] = xs
    smol_ref[...] = jnp.sum(xs * matts_ref[...], axis=1)


def _s11_body(p0_ref, p1_ref, mwT_ref, mattd_ref, out0_ref, od0_ref):
    out0 = jax.nn.relu(p0_ref[...] + p1_ref[...])
    out0_ref[...] = out0
    od = out0 @ mwT_ref[...]
    od0_ref[...] = jnp.sum(od * mattd_ref[...], axis=1)


def _s13_body(smol_ref, odb_ref, xs_ref, xsw_ref, w_ref):
    i = pl.program_id(0)
    wv = jnp.exp(_leaky(smol_ref[...] + odb_ref[...]))
    rowid = i * NODE_BLK + lax.broadcasted_iota(I32, (NODE_BLK,), 0)
    wv = jnp.where(rowid < N_NODES, wv, 0.0)
    w_ref[...] = wv
    xsw_ref[...] = xs_ref[...] * wv[:, None]


def _s15_body(h0_ref, h1_ref, d0_ref, d1_ref, out_ref,
              mb_ref, wihT_ref, whhT_ref, bih_ref, bhh_ref,
              mwT_ref, mattd_ref,
              out1_ref, od1_ref):
    hn = h0_ref[...] + h1_ref[...]
    dn = d0_ref[...] + d1_ref[...]
    hmean = jnp.where(dn[:, None] > 0, hn / (dn[:, None] + 1e-30), 0.0)
    h = _elu(hmean + mb_ref[...])
    out1 = jax.nn.relu(_gru_tc(h, out_ref[...], wihT_ref[...], whhT_ref[...],
                               bih_ref[...], bhh_ref[...]))
    out1_ref[...] = out1
    od = out1 @ mwT_ref[...]
    od1_ref[...] = jnp.sum(od * mattd_ref[...], axis=1)


def _s19_body(h0_ref, h1_ref, d0_ref, d1_ref, out_ref,
              mb_ref, wihT_ref, whhT_ref, bih_ref, bhh_ref,
              l2wT_ref, l2b_ref, y_ref):
    hn = h0_ref[...] + h1_ref[...]
    dn = d0_ref[...] + d1_ref[...]
    hmean = jnp.where(dn[:, None] > 0, hn / (dn[:, None] + 1e-30), 0.0)
    h = _elu(hmean + mb_ref[...])
    out2 = jax.nn.relu(_gru_tc(h, out_ref[...], wihT_ref[...], whhT_ref[...],
                               bih_ref[...], bhh_ref[...]))
    y = out2 @ l2wT_ref[...] + l2b_ref[...]
    y_ref[...] = y * jnp.tanh(jnp.log1p(jnp.exp(y)))


def _full(shape):
    return pl.BlockSpec(shape, lambda *_: tuple(0 for _ in shape))


def _rowblk(blk, width):
    return pl.BlockSpec((blk, width), lambda i: (i, 0))


def _rowblk1(blk):
    return pl.BlockSpec((blk,), lambda i: (i,))


def _rowblk_off(blk, width, off):
    return pl.BlockSpec((blk, width), lambda i: (i + off, 0))


def _rowblk1_off(blk, off):
    return pl.BlockSpec((blk,), lambda i: (i + off,))


def kernel(x, edge_index, edge_attr, batch_index, lin1_w, lin1_b, gate_lin1_w,
           gate_lin2_w, gate_att_l, gate_att_r, gate_bias, gru0_wih, gru0_whh,
           gru0_bih, gru0_bhh, conv1_w, conv1_att_src, conv1_att_dst,
           conv1_bias, gru1_wih, gru1_whh, gru1_bih, gru1_bhh, mol_w,
           mol_att_src, mol_att_dst, mol_bias, molgru_wih, molgru_whh,
           molgru_bih, molgru_bhh, lin2_w, lin2_b):
    src, dst = edge_index[0], edge_index[1]

    epad = (jnp.arange(E2 - N_EDGES, dtype=I32) * 17) % N_NODES
    srcb = jnp.concatenate([src, epad])
    dstb = jnp.concatenate([dst, epad])
    npad = (jnp.arange(NP - N_NODES, dtype=I32) * 17) % NUM_MOLS
    batchb = jnp.concatenate([batch_index, npad])
    ea_c = [jnp.pad(edge_attr[:, k], (0, E2 - N_EDGES)) for k in range(4)]
    x_p = jnp.pad(x, ((0, NP - N_NODES), (0, 0)))

    w1T = lin1_w.T
    b1 = lin1_b.reshape(1, HID)
    wxT = gate_lin1_w[:, :HID].T
    attr = gate_att_r.reshape(1, HID)
    attl = gate_att_l.reshape(1, HID)
    l2T = gate_lin2_w.T
    gb = gate_bias.reshape(1, HID)
    g0ihT, g0hhT = gru0_wih.T, gru0_whh.T
    g0bi, g0bh = gru0_bih.reshape(1, 3 * HID), gru0_bhh.reshape(1, 3 * HID)
    cwT = conv1_w.T
    atts = conv1_att_src.reshape(1, HID)
    attd = conv1_att_dst.reshape(1, HID)
    cb = conv1_bias.reshape(1, HID)
    g1ihT, g1hhT = gru1_wih.T, gru1_whh.T
    g1bi, g1bh = gru1_bih.reshape(1, 3 * HID), gru1_bhh.reshape(1, 3 * HID)
    mwT = mol_w.T
    matts = mol_att_src.reshape(1, HID)
    mattd = mol_att_dst.reshape(1, HID)
    mb = mol_bias.reshape(1, HID)
    mgihT, mghhT = molgru_wih.T, molgru_whh.T
    mgbi, mgbh = molgru_bih.reshape(1, 3 * HID), molgru_bhh.reshape(1, 3 * HID)
    l2wT = lin2_w.T
    l2b = lin2_b.reshape(1, OUT_CH)

    x1, g_tab, r_tab = pl.pallas_call(
        _s1_body,
        grid=(NB_N,),
        in_specs=[_rowblk(NODE_BLK, IN_CH), _full((IN_CH, HID)),
                  _full((1, HID)), _full((HID, HID)), _full((1, HID))],
        out_specs=[_rowblk(NODE_BLK, HID), _rowblk(NODE_BLK, HID),
                   _rowblk1(NODE_BLK)],
        out_shape=[jax.ShapeDtypeStruct((NP, HID), F32),
                   jax.ShapeDtypeStruct((NP, HID), F32),
                   jax.ShapeDtypeStruct((NP,), F32)],
    )(x_p, w1T, b1, wxT, attr)

    gsrc, rdst = _gate_gather(g_tab, r_tab, srcb, dstb)

    wm, w_e = pl.pallas_call(
        _s3_body,
        grid=(NB_E,),
        in_specs=[_rowblk(EDGE_BLK, HID),
                  _rowblk1(EDGE_BLK), _rowblk1(EDGE_BLK),
                  _rowblk1(EDGE_BLK), _rowblk1(EDGE_BLK),
                  _rowblk1(EDGE_BLK), _full((EDGE_DIM, HID)),
                  _full((1, HID))],
        out_specs=[_rowblk(EDGE_BLK, HID), _rowblk1(EDGE_BLK)],
        out_shape=[jax.ShapeDtypeStruct((E2, HID), F32),
                   jax.ShapeDtypeStruct((E2,), F32)],
    )(gsrc, ea_c[0], ea_c[1], ea_c[2], ea_c[3], rdst, gate_lin1_w[:, HID:].T,
      attl)

    hnum, den = _edge_scatter(dstb, wm, w_e)

    x2, xp_tab, s_tab, d_tab = pl.pallas_call(
        _s5_body,
        grid=(NB_N,),
        in_specs=[_rowblk(NODE_BLK, HID),
                  _rowblk1(NODE_BLK),
                  _rowblk(NODE_BLK, HID),
                  _full((HID, HID)), _full((1, HID)),
                  _full((HID, 3 * HID)), _full((HID, 3 * HID)),
                  _full((1, 3 * HID)), _full((1, 3 * HID)),
                  _full((HID, HID)), _full((1, HID)), _full((1, HID))],
        out_specs=[_rowblk(NODE_BLK, HID), _rowblk(NODE_BLK, HID),
                   _rowblk1(NODE_BLK), _rowblk1(NODE_BLK)],
        out_shape=[jax.ShapeDtypeStruct((NP, HID), F32),
                   jax.ShapeDtypeStruct((NP, HID), F32),
                   jax.ShapeDtypeStruct((NP,), F32),
                   jax.ShapeDtypeStruct((NP,), F32)],
    )(hnum, den, x1, l2T, gb, g0ihT, g0hhT, g0bi, g0bh,
      cwT, atts, attd)

    xps, ss, dd = _conv_gather(xp_tab, s_tab, d_tab, srcb, dstb)

    wxp, w2_e = pl.pallas_call(
        _s7_body,
        grid=(NB_E,),
        in_specs=[_rowblk1(EDGE_BLK), _rowblk1(EDGE_BLK),
                  _rowblk(EDGE_BLK, HID)],
        out_specs=[_rowblk(EDGE_BLK, HID), _rowblk1(EDGE_BLK)],
        out_shape=[jax.ShapeDtypeStruct((E2, HID), F32),
                   jax.ShapeDtypeStruct((E2,), F32)],
    )(ss, dd, xps)

    hnum2, den2 = _edge_scatter(dstb, wxp, w2_e)

    x3, xs_tab, smol = pl.pallas_call(
        _s9_body,
        grid=(NB_N,),
        in_specs=[_rowblk(NODE_BLK, HID),
                  _rowblk1(NODE_BLK),
                  _rowblk(NODE_BLK, HID),
                  _full((1, HID)),
                  _full((HID, 3 * HID)), _full((HID, 3 * HID)),
                  _full((1, 3 * HID)), _full((1, 3 * HID)),
                  _full((HID, HID)), _full((1, HID))],
        out_specs=[_rowblk(NODE_BLK, HID), _rowblk(NODE_BLK, HID),
                   _rowblk1(NODE_BLK)],
        out_shape=[jax.ShapeDtypeStruct((NP, HID), F32),
                   jax.ShapeDtypeStruct((NP, HID), F32),
                   jax.ShapeDtypeStruct((NP,), F32)],
    )(hnum2, den2, x2, cb, g1ihT, g1hhT, g1bi, g1bh, mwT, matts)

    pool = _pool_scatter(batchb, x3)

    out0, od0 = pl.pallas_call(
        _s11_body,
        grid=(1,),
        in_specs=[_rowblk(NUM_MOLS, HID), _rowblk_off(NUM_MOLS, HID, 1),
                  _full((HID, HID)), _full((1, HID))],
        out_specs=[_rowblk(NUM_MOLS, HID), _rowblk1(NUM_MOLS)],
        out_shape=[jax.ShapeDtypeStruct((NUM_MOLS, HID), F32),
                   jax.ShapeDtypeStruct((NUM_MOLS,), F32)],
    )(pool, pool, mwT, mattd)

    out_t = out0
    od_t = od0
    for t in range(2):
        odb = _mol_gather(od_t, batchb)
        xsw, w_m = pl.pallas_call(
            _s13_body,
            grid=(NB_N,),
            in_specs=[_rowblk1(NODE_BLK), _rowblk1(NODE_BLK),
                      _rowblk(NODE_BLK, HID)],
            out_specs=[_rowblk(NODE_BLK, HID), _rowblk1(NODE_BLK)],
            out_shape=[jax.ShapeDtypeStruct((NP, HID), F32),
                       jax.ShapeDtypeStruct((NP,), F32)],
        )(smol, odb, xs_tab)
        mnum, mden = _mol_scatter(batchb, xsw, w_m)
        if t == 0:
            out_t, od_t = pl.pallas_call(
                _s15_body,
                grid=(1,),
                in_specs=[_rowblk(NUM_MOLS, HID),
                          _rowblk_off(NUM_MOLS, HID, 1),
                          _rowblk1(NUM_MOLS), _rowblk1_off(NUM_MOLS, 1),
                          _rowblk(NUM_MOLS, HID),
                          _full((1, HID)),
                          _full((HID, 3 * HID)), _full((HID, 3 * HID)),
                          _full((1, 3 * HID)), _full((1, 3 * HID)),
                          _full((HID, HID)), _full((1, HID))],
                out_specs=[_rowblk(NUM_MOLS, HID), _rowblk1(NUM_MOLS)],
                out_shape=[jax.ShapeDtypeStruct((NUM_MOLS, HID), F32),
                           jax.ShapeDtypeStruct((NUM_MOLS,), F32)],
            )(mnum, mnum, mden, mden, out_t, mb, mgihT, mghhT, mgbi, mgbh,
              mwT, mattd)
        else:
            y = pl.pallas_call(
                _s19_body,
                grid=(1,),
                in_specs=[_rowblk(NUM_MOLS, HID),
                          _rowblk_off(NUM_MOLS, HID, 1),
                          _rowblk1(NUM_MOLS), _rowblk1_off(NUM_MOLS, 1),
                          _rowblk(NUM_MOLS, HID),
                          _full((1, HID)),
                          _full((HID, 3 * HID)), _full((HID, 3 * HID)),
                          _full((1, 3 * HID)), _full((1, 3 * HID)),
                          _full((HID, OUT_CH)), _full((1, OUT_CH))],
                out_specs=_rowblk(NUM_MOLS, OUT_CH),
                out_shape=jax.ShapeDtypeStruct((NUM_MOLS, OUT_CH), F32),
            )(mnum, mnum, mden, mden, out_t, mb, mgihT, mghhT, mgbi, mgbh,
              l2wT, l2b)
    return y
